# baseline (speedup 1.0000x reference)
import functools
import jax
import jax.numpy as jnp
from jax import lax
from jax.experimental import pallas as pl
from jax.experimental.pallas import tpu as pltpu
from jax.experimental.pallas import tpu_sc as plsc

_N = 10000
_E = 320000
_F = 128
_NCLASS = 40
_NLAYERS = 3

_NP = 10240
_SUB = 16
_CH = 128
_NCH = 158
_EP = _SUB * _NCH * _CH
_RPS = _NP // _SUB

_R = 640
_GRID = _NP // _R

_sc_mesh = plsc.VectorSubcoreMesh(core_axis_name="c", subcore_axis_name="s")


@functools.partial(
    pl.kernel,
    out_type=(
        jax.ShapeDtypeStruct((_NP,), jnp.float32),
        jax.ShapeDtypeStruct((_NP,), jnp.float32),
    ),
    mesh=_sc_mesh,
    scratch_types=[
        pltpu.VMEM((_NCH, _CH), jnp.int32),
        pltpu.VMEM((_CH,), jnp.float32),
        pltpu.VMEM_SHARED((_NP + 256,), jnp.float32),
    ],
)
def _deg_kernel(ers_hbm, zfull_hbm, degout_hbm, indeg_hbm, idx_v, ones_v, dsh):
    c = lax.axis_index("c")
    s = lax.axis_index("s")
    rs = pl.ds(s * _RPS, _RPS)

    @pl.when(s == 0)
    def _():
        pltpu.sync_copy(zfull_hbm, dsh.at[pl.ds(0, _NP)])

    pltpu.sync_copy(ers_hbm.at[c, s], idx_v)
    ones = jnp.full((16,), 1.0, jnp.float32)
    for k in range(_CH // 16):
        ones_v[pl.ds(k * 16, 16)] = ones
    plsc.subcore_barrier()

    def body(j, carry):
        pltpu.sync_copy(ones_v, dsh.at[idx_v.at[j]], add=True)
        return carry

    lax.fori_loop(0, _NCH, body, 0)
    plsc.subcore_barrier()

    @pl.when(c == 0)
    def _():
        pltpu.sync_copy(dsh.at[rs], degout_hbm.at[rs])

    @pl.when(c == 1)
    def _():
        pltpu.sync_copy(dsh.at[rs], indeg_hbm.at[rs])


_HALF = _NP // 2
_ACCN = 5248
_WPS = _HALF // _SUB
_ZPS = _ACCN // _SUB
def _chunks(total, ch):
    full, rem = divmod(total, ch)
    return (ch,) * full + ((rem,) if rem else ())


_ZCH = _chunks(_ZPS, _CH)
_WCH = _chunks(_WPS, _CH)
_NPAIR = _NCH // 2


@functools.partial(
    pl.kernel,
    out_type=(
        jax.ShapeDtypeStruct((2, _NP, _F), jnp.float32),
        jax.ShapeDtypeStruct((_NP,), jnp.float32),
    ),
    mesh=_sc_mesh,
    scratch_types=[
        pltpu.VMEM((_NCH, _CH), jnp.int32),
        pltpu.VMEM((_NCH, _CH), jnp.int32),
        pltpu.VMEM((_CH,), jnp.int32),
        pltpu.VMEM((_CH,), jnp.int32),
        pltpu.VMEM((_CH, _F), jnp.float32),
        pltpu.VMEM((_CH, _F), jnp.float32),
        pltpu.VMEM((_NP,), jnp.float32),
        pltpu.VMEM((_CH,), jnp.float32),
        pltpu.VMEM((_CH,), jnp.float32),
        pltpu.VMEM_SHARED((_ACCN, _F), jnp.float32),
        pltpu.VMEM_SHARED((_ACCN,), jnp.float32),
        pltpu.SemaphoreType.DMA,
        pltpu.SemaphoreType.DMA,
    ],
    compiler_params=pltpu.CompilerParams(needs_layout_passes=False),
)
def _edge_kernel(h_hbm, g_hbm, sq_hbm, erg_hbm, ers_hbm, z128_hbm, zfull_hbm,
                 out128_hbm, t_hbm,
                 idx_g, idx_s, idx_c0, idx_c1, rows0, rows1, sqv, sv0, sv1,
                 acc, tsh, sem0, sem1):
    c = lax.axis_index("c")
    s = lax.axis_index("s")
    base = c * _HALF

    def zero_acc():
        off = 0
        for n in _ZCH:
            pltpu.sync_copy(rows0.at[pl.ds(0, n)],
                            acc.at[pl.ds(s * _ZPS + off, n)])
            off += n

    def writeout(d):
        off = 0
        for n in _WCH:
            pltpu.sync_copy(acc.at[pl.ds(s * _WPS + off, n)],
                            rows0.at[pl.ds(0, n)])
            pltpu.sync_copy(rows0.at[pl.ds(0, n)],
                            out128_hbm.at[d, pl.ds(base + s * _WPS + off, n)])
            off += n

    pltpu.sync_copy(z128_hbm, rows0)
    zero_acc()

    pltpu.sync_copy(sq_hbm, sqv)

    @pl.when(s == 0)
    def _():
        pltpu.sync_copy(zfull_hbm.at[pl.ds(0, _ACCN)], tsh)

    def clamp_store(sr, idx_c):
        for k in range(_CH // 16):
            si = sr[pl.ds(k * 16, 16)]
            lo = si - base
            ok = jnp.logical_and(lo >= 0, lo < _HALF)
            idx_c[pl.ds(k * 16, 16)] = jnp.where(ok, lo, _HALF)

    def sv_gather(gr, sv):
        for k in range(_CH // 16):
            gi = gr[pl.ds(k * 16, 16)]
            sv[pl.ds(k * 16, 16)] = plsc.load_gather(sqv, [gi])

    pltpu.sync_copy(erg_hbm.at[1, s], idx_g)
    pltpu.sync_copy(ers_hbm.at[0, s], idx_s)
    plsc.subcore_barrier()

    def body_a(p, carry):
        j0 = 2 * p
        j1 = j0 + 1
        cp0 = pltpu.async_copy(h_hbm.at[idx_g.at[j0]], rows0, sem0)
        cp1 = pltpu.async_copy(h_hbm.at[idx_g.at[j1]], rows1, sem1)
        clamp_store(idx_s.at[j0], idx_c0)
        sv_gather(idx_g.at[j0], sv0)
        cp0.wait()
        pltpu.sync_copy(rows0, acc.at[idx_c0], add=True)
        pltpu.sync_copy(sv0, tsh.at[idx_c0], add=True)
        clamp_store(idx_s.at[j1], idx_c1)
        sv_gather(idx_g.at[j1], sv1)
        cp1.wait()
        pltpu.sync_copy(rows1, acc.at[idx_c1], add=True)
        pltpu.sync_copy(sv1, tsh.at[idx_c1], add=True)
        return carry

    lax.fori_loop(0, _NPAIR, body_a, 0)
    plsc.subcore_barrier()

    writeout(0)
    pltpu.sync_copy(tsh.at[pl.ds(s * _WPS, _WPS)], sqv.at[pl.ds(0, _WPS)])
    pltpu.sync_copy(sqv.at[pl.ds(0, _WPS)],
                    t_hbm.at[pl.ds(base + s * _WPS, _WPS)])
    pltpu.sync_copy(z128_hbm, rows0)
    zero_acc()

    pltpu.sync_copy(erg_hbm.at[0, s], idx_g)
    pltpu.sync_copy(ers_hbm.at[1, s], idx_s)
    plsc.subcore_barrier()

    def body_b(p, carry):
        j0 = 2 * p
        j1 = j0 + 1
        cp0 = pltpu.async_copy(g_hbm.at[idx_g.at[j0]], rows0, sem0)
        cp1 = pltpu.async_copy(g_hbm.at[idx_g.at[j1]], rows1, sem1)
        clamp_store(idx_s.at[j0], idx_c0)
        cp0.wait()
        pltpu.sync_copy(rows0, acc.at[idx_c0], add=True)
        clamp_store(idx_s.at[j1], idx_c1)
        cp1.wait()
        pltpu.sync_copy(rows1, acc.at[idx_c1], add=True)
        return carry

    lax.fori_loop(0, _NPAIR, body_b, 0)
    plsc.subcore_barrier()
    writeout(1)


def _pre_body(x_ref, encw_ref, encb_ref, skipw_ref, convw_ref, indeg_ref,
              h_ref, xs_ref, g_ref, sq_ref, gm_ref):
    i = pl.program_id(0)
    x = x_ref[...]
    h = jnp.maximum(x @ encw_ref[...] + encb_ref[...], 0.0)
    h_ref[...] = h
    xs_ref[...] = x @ skipw_ref[...]
    dinv = lax.rsqrt(indeg_ref[...] + 1.0)
    g_ref[...] = (h @ convw_ref[...]) * dinv
    sq_ref[...] = jnp.sum(h * h, axis=1, keepdims=True)

    @pl.when(i == 0)
    def _():
        gm_ref[...] = jnp.zeros_like(gm_ref)

    rid = i * _R + lax.broadcasted_iota(jnp.int32, (_R, 1), 0)
    w = jnp.where(rid < _N, 1.0 / _N, 0.0)
    gm_ref[...] += jnp.sum(h * w, axis=0, keepdims=True)


_pre_call = pl.pallas_call(
    _pre_body,
    grid=(_GRID,),
    in_specs=[
        pl.BlockSpec((_R, _F), lambda i: (i, 0)),
        pl.BlockSpec((_F, _F), lambda i: (0, 0)),
        pl.BlockSpec((1, _F), lambda i: (0, 0)),
        pl.BlockSpec((_F, _F), lambda i: (0, 0)),
        pl.BlockSpec((_F, _F), lambda i: (0, 0)),
        pl.BlockSpec((_R, 1), lambda i: (i, 0)),
    ],
    out_specs=[
        pl.BlockSpec((_R, _F), lambda i: (i, 0)),
        pl.BlockSpec((_R, _F), lambda i: (i, 0)),
        pl.BlockSpec((_R, _F), lambda i: (i, 0)),
        pl.BlockSpec((_R, 1), lambda i: (i, 0)),
        pl.BlockSpec((1, _F), lambda i: (0, 0)),
    ],
    out_shape=[
        jax.ShapeDtypeStruct((_NP, _F), jnp.float32),
        jax.ShapeDtypeStruct((_NP, _F), jnp.float32),
        jax.ShapeDtypeStruct((_NP, _F), jnp.float32),
        jax.ShapeDtypeStruct((_NP, 1), jnp.float32),
        jax.ShapeDtypeStruct((1, _F), jnp.float32),
    ],
)


def _layer_body(h_ref, xs_ref, g_ref, o128_ref, t_ref, od_ref, id_ref,
                gm_ref, convw_ref, convb_ref,
                hn_ref, gn_ref, sqn_ref, gmn_ref):
    i = pl.program_id(0)
    h = h_ref[...]
    s_agg = o128_ref[0]
    agg_raw = o128_ref[1]
    t = t_ref[...]
    outdeg = od_ref[...]
    dinv = lax.rsqrt(id_ref[...] + 1.0)

    x_agg = jnp.maximum(dinv * (agg_raw + g_ref[...]) + convb_ref[...], 0.0)

    sq = jnp.sum(h * h, axis=1, keepdims=True)
    hs = jnp.sum(h * s_agg, axis=1, keepdims=True)
    gamma = (outdeg * sq - 2.0 * hs + t) / (outdeg + 1e-10)
    gs = jnp.tanh(gamma)

    d = h - gm_ref[...]
    d2 = d * d
    dq = jnp.sum(d2 * jnp.sqrt(jnp.sqrt(d2)), axis=1, keepdims=True)
    gq = 1.0 - jnp.tanh(dq)

    hn = (h + gs * x_agg + gq * xs_ref[...]) / (1.0 + gs + gq)
    hn_ref[...] = hn
    gn_ref[...] = (hn @ convw_ref[...]) * dinv
    sqn_ref[...] = jnp.sum(hn * hn, axis=1, keepdims=True)

    @pl.when(i == 0)
    def _():
        gmn_ref[...] = jnp.zeros_like(gmn_ref)

    rid = i * _R + lax.broadcasted_iota(jnp.int32, (_R, 1), 0)
    w = jnp.where(rid < _N, 1.0 / _N, 0.0)
    gmn_ref[...] += jnp.sum(hn * w, axis=0, keepdims=True)


_layer_call = pl.pallas_call(
    _layer_body,
    grid=(_GRID,),
    in_specs=[
        pl.BlockSpec((_R, _F), lambda i: (i, 0)),
        pl.BlockSpec((_R, _F), lambda i: (i, 0)),
        pl.BlockSpec((_R, _F), lambda i: (i, 0)),
        pl.BlockSpec((2, _R, _F), lambda i: (0, i, 0)),
        pl.BlockSpec((_R, 1), lambda i: (i, 0)),
        pl.BlockSpec((_R, 1), lambda i: (i, 0)),
        pl.BlockSpec((_R, 1), lambda i: (i, 0)),
        pl.BlockSpec((1, _F), lambda i: (0, 0)),
        pl.BlockSpec((_F, _F), lambda i: (0, 0)),
        pl.BlockSpec((1, _F), lambda i: (0, 0)),
    ],
    out_specs=[
        pl.BlockSpec((_R, _F), lambda i: (i, 0)),
        pl.BlockSpec((_R, _F), lambda i: (i, 0)),
        pl.BlockSpec((_R, 1), lambda i: (i, 0)),
        pl.BlockSpec((1, _F), lambda i: (0, 0)),
    ],
    out_shape=[
        jax.ShapeDtypeStruct((_NP, _F), jnp.float32),
        jax.ShapeDtypeStruct((_NP, _F), jnp.float32),
        jax.ShapeDtypeStruct((_NP, 1), jnp.float32),
        jax.ShapeDtypeStruct((1, _F), jnp.float32),
    ],
)


def _post_body(h_ref, decw_ref, decb_ref, out_ref):
    out_ref[...] = h_ref[...] @ decw_ref[...] + decb_ref[...]


_post_call = pl.pallas_call(
    _post_body,
    grid=(_GRID,),
    in_specs=[
        pl.BlockSpec((_R, _F), lambda i: (i, 0)),
        pl.BlockSpec((_F, _NCLASS), lambda i: (0, 0)),
        pl.BlockSpec((1, _NCLASS), lambda i: (0, 0)),
    ],
    out_specs=pl.BlockSpec((_R, _NCLASS), lambda i: (i, 0)),
    out_shape=jax.ShapeDtypeStruct((_NP, _NCLASS), jnp.float32),
)


def kernel(x, edge_index, enc_W, enc_b, skip_W, conv_W, conv_b, dec_W, dec_b):
    pad = _EP - _E
    erg = jnp.pad(edge_index, ((0, 0), (0, pad))).reshape(
        2, _SUB, _NCH, _CH)
    ers = jnp.pad(edge_index, ((0, 0), (0, pad)),
                  constant_values=_NP).reshape(2, _SUB, _NCH, _CH)
    zfull = jnp.zeros((_NP,), jnp.float32)
    z128 = jnp.zeros((_CH, _F), jnp.float32)
    enc_b2 = enc_b.reshape(1, _F)
    conv_b2 = conv_b.reshape(1, _F)
    dec_b2 = dec_b.reshape(1, _NCLASS)
    xp = jnp.pad(x, ((0, _NP - _N), (0, 0)))

    outdeg, indeg = _deg_kernel(ers, zfull)
    outdeg2 = outdeg.reshape(_NP, 1)
    indeg2 = indeg.reshape(_NP, 1)

    h, xs, g, sq, gm = _pre_call(xp, enc_W, enc_b2, skip_W, conv_W, indeg2)

    def lbody(_, carry):
        h, g, sq, gm = carry
        out128, t = _edge_kernel(h, g, sq.reshape(_NP), erg, ers,
                                 z128, zfull)
        return _layer_call(h, xs, g, out128, t.reshape(_NP, 1),
                           outdeg2, indeg2, gm, conv_W, conv_b2)

    h, g, sq, gm = lax.fori_loop(0, _NLAYERS, lbody, (h, g, sq, gm))
    return _post_call(h, dec_W, dec_b2)[:_N]


# speedup vs baseline: 1.3800x; 1.3800x over previous
import functools
import jax
import jax.numpy as jnp
from jax import lax
from jax.experimental import pallas as pl
from jax.experimental.pallas import tpu as pltpu
from jax.experimental.pallas import tpu_sc as plsc

_N = 10000
_E = 320000
_F = 128
_NCLASS = 40
_NLAYERS = 3

_NP = 10240
_SUB = 16
_CH = 80
_NCH = 250
_EP = _SUB * _NCH * _CH
_RPS = _NP // _SUB

_R = 640
_GRID = _NP // _R

_sc_mesh = plsc.VectorSubcoreMesh(core_axis_name="c", subcore_axis_name="s")


@functools.partial(
    pl.kernel,
    out_type=(
        jax.ShapeDtypeStruct((_NP,), jnp.float32),
        jax.ShapeDtypeStruct((_NP,), jnp.float32),
    ),
    mesh=_sc_mesh,
    scratch_types=[
        pltpu.VMEM((_NCH, _CH), jnp.int32),
        pltpu.VMEM((_CH,), jnp.float32),
        pltpu.VMEM_SHARED((_NP + 256,), jnp.float32),
    ],
)
def _deg_kernel(ers_hbm, zfull_hbm, degout_hbm, indeg_hbm, idx_v, ones_v, dsh):
    c = lax.axis_index("c")
    s = lax.axis_index("s")
    rs = pl.ds(s * _RPS, _RPS)

    @pl.when(s == 0)
    def _():
        pltpu.sync_copy(zfull_hbm, dsh.at[pl.ds(0, _NP)])

    pltpu.sync_copy(ers_hbm.at[c, s], idx_v)
    ones = jnp.full((16,), 1.0, jnp.float32)
    for k in range(_CH // 16):
        ones_v[pl.ds(k * 16, 16)] = ones
    plsc.subcore_barrier()

    def body(j, carry):
        pltpu.sync_copy(ones_v, dsh.at[idx_v.at[j]], add=True)
        return carry

    lax.fori_loop(0, _NCH, body, 0)
    plsc.subcore_barrier()

    @pl.when(c == 0)
    def _():
        pltpu.sync_copy(dsh.at[rs], degout_hbm.at[rs])

    @pl.when(c == 1)
    def _():
        pltpu.sync_copy(dsh.at[rs], indeg_hbm.at[rs])


_HALF = _NP // 2
_ACCN = 5248
_WPS = _HALF // _SUB
_ZPS = _ACCN // _SUB
def _chunks(total, ch):
    full, rem = divmod(total, ch)
    return (ch,) * full + ((rem,) if rem else ())


_ZCH = _chunks(_ZPS, _CH)
_WCH = _chunks(_WPS, _CH)
_NPAIR = _NCH // 2


@functools.partial(
    pl.kernel,
    out_type=(
        jax.ShapeDtypeStruct((2, _NP, _F), jnp.float32),
        jax.ShapeDtypeStruct((_NP,), jnp.float32),
    ),
    mesh=_sc_mesh,
    scratch_types=[
        pltpu.VMEM((_NCH, _CH), jnp.int32),
        pltpu.VMEM((_NCH, _CH), jnp.int32),
        pltpu.VMEM((_CH,), jnp.int32),
        pltpu.VMEM((_CH, _F), jnp.float32),
        pltpu.VMEM((_NP,), jnp.float32),
        pltpu.VMEM((_CH,), jnp.float32),
        pltpu.VMEM_SHARED((_ACCN, _F), jnp.float32),
        pltpu.VMEM_SHARED((_ACCN,), jnp.float32),
        pltpu.SemaphoreType.DMA,
    ],
    compiler_params=pltpu.CompilerParams(needs_layout_passes=False),
)
def _edge_kernel(h_hbm, g_hbm, sq_hbm, erg_hbm, ers_hbm, z128_hbm, zfull_hbm,
                 out128_hbm, t_hbm,
                 idx_g, idx_s, idx_c, rows, sqv, svals, acc, tsh, sem0):
    c = lax.axis_index("c")
    s = lax.axis_index("s")
    base = c * _HALF

    def zero_acc():
        off = 0
        for n in _ZCH:
            pltpu.sync_copy(rows.at[pl.ds(0, n)],
                            acc.at[pl.ds(s * _ZPS + off, n)])
            off += n

    def writeout(d):
        off = 0
        for n in _WCH:
            pltpu.sync_copy(acc.at[pl.ds(s * _WPS + off, n)],
                            rows.at[pl.ds(0, n)])
            pltpu.sync_copy(rows.at[pl.ds(0, n)],
                            out128_hbm.at[d, pl.ds(base + s * _WPS + off, n)])
            off += n

    pltpu.sync_copy(z128_hbm, rows)
    zero_acc()

    pltpu.sync_copy(sq_hbm, sqv)

    @pl.when(s == 0)
    def _():
        pltpu.sync_copy(zfull_hbm.at[pl.ds(0, _ACCN)], tsh)

    lanes = lax.iota(jnp.int32, 16)

    def clamp_store(sr):
        for k in range(_CH // 16):
            si = sr[pl.ds(k * 16, 16)]
            lo = si - base
            ok = jnp.logical_and(lo >= 0, lo < _HALF)
            idx_c[pl.ds(k * 16, 16)] = jnp.where(ok, lo,
                                                 _HALF + k * 16 + lanes)

    def sv_gather(gr):
        for k in range(_CH // 16):
            gi = gr[pl.ds(k * 16, 16)]
            svals[pl.ds(k * 16, 16)] = plsc.load_gather(sqv, [gi])

    pltpu.sync_copy(erg_hbm.at[1, s], idx_g)
    pltpu.sync_copy(ers_hbm.at[0, s], idx_s)
    plsc.subcore_barrier()

    def body_a(j, carry):
        cp = pltpu.async_copy(h_hbm.at[idx_g.at[j]], rows, sem0)
        clamp_store(idx_s.at[j])
        sv_gather(idx_g.at[j])
        cp.wait()
        pltpu.sync_copy(rows, acc.at[idx_c], add=True)
        pltpu.sync_copy(svals, tsh.at[idx_c], add=True)
        return carry

    lax.fori_loop(0, _NCH, body_a, 0)
    plsc.subcore_barrier()

    writeout(0)
    pltpu.sync_copy(tsh.at[pl.ds(s * _WPS, _WPS)], sqv.at[pl.ds(0, _WPS)])
    pltpu.sync_copy(sqv.at[pl.ds(0, _WPS)],
                    t_hbm.at[pl.ds(base + s * _WPS, _WPS)])
    pltpu.sync_copy(z128_hbm, rows)
    zero_acc()

    pltpu.sync_copy(erg_hbm.at[0, s], idx_g)
    pltpu.sync_copy(ers_hbm.at[1, s], idx_s)
    plsc.subcore_barrier()

    def body_b(j, carry):
        cp = pltpu.async_copy(g_hbm.at[idx_g.at[j]], rows, sem0)
        clamp_store(idx_s.at[j])
        cp.wait()
        pltpu.sync_copy(rows, acc.at[idx_c], add=True)
        return carry

    lax.fori_loop(0, _NCH, body_b, 0)
    plsc.subcore_barrier()
    writeout(1)


def _pre_body(x_ref, encw_ref, encb_ref, skipw_ref, convw_ref, indeg_ref,
              h_ref, xs_ref, g_ref, sq_ref, gm_ref):
    i = pl.program_id(0)
    x = x_ref[...]
    h = jnp.maximum(x @ encw_ref[...] + encb_ref[...], 0.0)
    h_ref[...] = h
    xs_ref[...] = x @ skipw_ref[...]
    dinv = lax.rsqrt(indeg_ref[...] + 1.0)
    g_ref[...] = (h @ convw_ref[...]) * dinv
    sq_ref[...] = jnp.sum(h * h, axis=1, keepdims=True)

    @pl.when(i == 0)
    def _():
        gm_ref[...] = jnp.zeros_like(gm_ref)

    rid = i * _R + lax.broadcasted_iota(jnp.int32, (_R, 1), 0)
    w = jnp.where(rid < _N, 1.0 / _N, 0.0)
    gm_ref[...] += jnp.sum(h * w, axis=0, keepdims=True)


_pre_call = pl.pallas_call(
    _pre_body,
    grid=(_GRID,),
    in_specs=[
        pl.BlockSpec((_R, _F), lambda i: (i, 0)),
        pl.BlockSpec((_F, _F), lambda i: (0, 0)),
        pl.BlockSpec((1, _F), lambda i: (0, 0)),
        pl.BlockSpec((_F, _F), lambda i: (0, 0)),
        pl.BlockSpec((_F, _F), lambda i: (0, 0)),
        pl.BlockSpec((_R, 1), lambda i: (i, 0)),
    ],
    out_specs=[
        pl.BlockSpec((_R, _F), lambda i: (i, 0)),
        pl.BlockSpec((_R, _F), lambda i: (i, 0)),
        pl.BlockSpec((_R, _F), lambda i: (i, 0)),
        pl.BlockSpec((_R, 1), lambda i: (i, 0)),
        pl.BlockSpec((1, _F), lambda i: (0, 0)),
    ],
    out_shape=[
        jax.ShapeDtypeStruct((_NP, _F), jnp.float32),
        jax.ShapeDtypeStruct((_NP, _F), jnp.float32),
        jax.ShapeDtypeStruct((_NP, _F), jnp.float32),
        jax.ShapeDtypeStruct((_NP, 1), jnp.float32),
        jax.ShapeDtypeStruct((1, _F), jnp.float32),
    ],
)


def _layer_body(h_ref, xs_ref, g_ref, o128_ref, t_ref, od_ref, id_ref,
                gm_ref, convw_ref, convb_ref,
                hn_ref, gn_ref, sqn_ref, gmn_ref):
    i = pl.program_id(0)
    h = h_ref[...]
    s_agg = o128_ref[0]
    agg_raw = o128_ref[1]
    t = t_ref[...]
    outdeg = od_ref[...]
    dinv = lax.rsqrt(id_ref[...] + 1.0)

    x_agg = jnp.maximum(dinv * (agg_raw + g_ref[...]) + convb_ref[...], 0.0)

    sq = jnp.sum(h * h, axis=1, keepdims=True)
    hs = jnp.sum(h * s_agg, axis=1, keepdims=True)
    gamma = (outdeg * sq - 2.0 * hs + t) / (outdeg + 1e-10)
    gs = jnp.tanh(gamma)

    d = h - gm_ref[...]
    d2 = d * d
    dq = jnp.sum(d2 * jnp.sqrt(jnp.sqrt(d2)), axis=1, keepdims=True)
    gq = 1.0 - jnp.tanh(dq)

    hn = (h + gs * x_agg + gq * xs_ref[...]) / (1.0 + gs + gq)
    hn_ref[...] = hn
    gn_ref[...] = (hn @ convw_ref[...]) * dinv
    sqn_ref[...] = jnp.sum(hn * hn, axis=1, keepdims=True)

    @pl.when(i == 0)
    def _():
        gmn_ref[...] = jnp.zeros_like(gmn_ref)

    rid = i * _R + lax.broadcasted_iota(jnp.int32, (_R, 1), 0)
    w = jnp.where(rid < _N, 1.0 / _N, 0.0)
    gmn_ref[...] += jnp.sum(hn * w, axis=0, keepdims=True)


_layer_call = pl.pallas_call(
    _layer_body,
    grid=(_GRID,),
    in_specs=[
        pl.BlockSpec((_R, _F), lambda i: (i, 0)),
        pl.BlockSpec((_R, _F), lambda i: (i, 0)),
        pl.BlockSpec((_R, _F), lambda i: (i, 0)),
        pl.BlockSpec((2, _R, _F), lambda i: (0, i, 0)),
        pl.BlockSpec((_R, 1), lambda i: (i, 0)),
        pl.BlockSpec((_R, 1), lambda i: (i, 0)),
        pl.BlockSpec((_R, 1), lambda i: (i, 0)),
        pl.BlockSpec((1, _F), lambda i: (0, 0)),
        pl.BlockSpec((_F, _F), lambda i: (0, 0)),
        pl.BlockSpec((1, _F), lambda i: (0, 0)),
    ],
    out_specs=[
        pl.BlockSpec((_R, _F), lambda i: (i, 0)),
        pl.BlockSpec((_R, _F), lambda i: (i, 0)),
        pl.BlockSpec((_R, 1), lambda i: (i, 0)),
        pl.BlockSpec((1, _F), lambda i: (0, 0)),
    ],
    out_shape=[
        jax.ShapeDtypeStruct((_NP, _F), jnp.float32),
        jax.ShapeDtypeStruct((_NP, _F), jnp.float32),
        jax.ShapeDtypeStruct((_NP, 1), jnp.float32),
        jax.ShapeDtypeStruct((1, _F), jnp.float32),
    ],
)


def _post_body(h_ref, decw_ref, decb_ref, out_ref):
    out_ref[...] = h_ref[...] @ decw_ref[...] + decb_ref[...]


_post_call = pl.pallas_call(
    _post_body,
    grid=(_GRID,),
    in_specs=[
        pl.BlockSpec((_R, _F), lambda i: (i, 0)),
        pl.BlockSpec((_F, _NCLASS), lambda i: (0, 0)),
        pl.BlockSpec((1, _NCLASS), lambda i: (0, 0)),
    ],
    out_specs=pl.BlockSpec((_R, _NCLASS), lambda i: (i, 0)),
    out_shape=jax.ShapeDtypeStruct((_NP, _NCLASS), jnp.float32),
)


def kernel(x, edge_index, enc_W, enc_b, skip_W, conv_W, conv_b, dec_W, dec_b):
    pad = _EP - _E
    erg = jnp.pad(edge_index, ((0, 0), (0, pad))).reshape(
        2, _SUB, _NCH, _CH)
    ers = jnp.pad(edge_index, ((0, 0), (0, pad)),
                  constant_values=_NP).reshape(2, _SUB, _NCH, _CH)
    zfull = jnp.zeros((_NP,), jnp.float32)
    z128 = jnp.zeros((_CH, _F), jnp.float32)
    enc_b2 = enc_b.reshape(1, _F)
    conv_b2 = conv_b.reshape(1, _F)
    dec_b2 = dec_b.reshape(1, _NCLASS)
    xp = jnp.pad(x, ((0, _NP - _N), (0, 0)))

    outdeg, indeg = _deg_kernel(ers, zfull)
    outdeg2 = outdeg.reshape(_NP, 1)
    indeg2 = indeg.reshape(_NP, 1)

    h, xs, g, sq, gm = _pre_call(xp, enc_W, enc_b2, skip_W, conv_W, indeg2)

    def lbody(_, carry):
        h, g, sq, gm = carry
        out128, t = _edge_kernel(h, g, sq.reshape(_NP), erg, ers,
                                 z128, zfull)
        return _layer_call(h, xs, g, out128, t.reshape(_NP, 1),
                           outdeg2, indeg2, gm, conv_W, conv_b2)

    h, g, sq, gm = lax.fori_loop(0, _NLAYERS, lbody, (h, g, sq, gm))
    return _post_call(h, dec_W, dec_b2)[:_N]
